# Initial kernel scaffold; baseline (speedup 1.0000x reference)
import jax
import jax.numpy as jnp
from jax.experimental import pallas as pl


def kernel(feat_1d, coords_1d, W1, b1, Wr, br, W2, b2):
    raise NotImplementedError("write your pallas kernel here")


# speedup vs baseline: 5.8924x; 5.8924x over previous
import functools

import jax
import jax.numpy as jnp
from jax import lax
from jax.experimental import pallas as pl
from jax.experimental.pallas import tpu as pltpu
from jax.experimental.pallas import tpu_sc as plsc

_B, _C, _L, _N = 8, 128, 8192, 8192
_K = 5
_H = 64
_BN = _B * _N
_BL = _B * _L
_NW = 32
_PPW = _BN // _NW
_CHA = 128
_PB = 2048

_mesh = plsc.VectorSubcoreMesh(core_axis_name="c", subcore_axis_name="s")
_sc_params = pltpu.CompilerParams(needs_layout_passes=False)


def _worker_base():
    wid = lax.axis_index("s") * 2 + lax.axis_index("c")
    return wid * _PPW


def _anchor_ix(xv):
    return jnp.clip((xv + 1.0) * 0.5 * (_L - 1), 0.0, float(_L - 1))


@functools.partial(
    pl.kernel,
    out_type=(
        jax.ShapeDtypeStruct((_BN, _C), jnp.float32),
        jax.ShapeDtypeStruct((_BN, _C), jnp.float32),
    ),
    mesh=_mesh,
    scratch_types=[
        pltpu.VMEM((_PPW,), jnp.float32),
        pltpu.VMEM((_CHA,), jnp.int32),
        pltpu.VMEM((_CHA,), jnp.int32),
        pltpu.VMEM((_CHA, _C), jnp.float32),
        pltpu.VMEM((_CHA, _C), jnp.float32),
        pltpu.SemaphoreType.DMA,
        pltpu.SemaphoreType.DMA,
    ],
    compiler_params=_sc_params,
)
def _stage_a(coords_hbm, featT_hbm, f0_hbm, f1_hbm,
             coords_v, idx0_v, idx1_v, buf0_v, buf1_v, sem0, sem1):
    base = _worker_base()
    boff = (base // _N) * _L
    pltpu.sync_copy(coords_hbm.at[pl.ds(base, _PPW)], coords_v)

    def round_body(g, carry):
        rbase = g * _CHA
        for i in range(_CHA // 16):
            xv = coords_v[pl.ds(rbase + i * 16, 16)]
            ix = _anchor_ix(xv)
            x0 = ix.astype(jnp.int32)
            x1 = jnp.minimum(x0 + 1, _L - 1)
            idx0_v[pl.ds(i * 16, 16)] = x0 + boff
            idx1_v[pl.ds(i * 16, 16)] = x1 + boff
        cp0 = pltpu.async_copy(featT_hbm.at[idx0_v], buf0_v, sem0)
        cp1 = pltpu.async_copy(featT_hbm.at[idx1_v], buf1_v, sem1)
        cp0.wait()
        cp1.wait()
        pltpu.sync_copy(buf0_v, f0_hbm.at[pl.ds(base + rbase, _CHA)])
        pltpu.sync_copy(buf1_v, f1_hbm.at[pl.ds(base + rbase, _CHA)])
        return carry

    lax.fori_loop(0, _PPW // _CHA, round_body, 0)


def _router_body(f0_ref, f1_ref, xc_ref, w1a_ref, w1c_ref, b1_ref,
                 wr_ref, br_ref, w2_ref, b2_ref, gb_ref, ww_ref):
    x = xc_ref[...]
    ixa = _anchor_ix(x)
    x0f = jnp.floor(ixa)
    wa = ixa - x0f
    fa = f0_ref[...] * (1.0 - wa) + f1_ref[...] * wa

    h = jnp.dot(fa, w1a_ref[...], preferred_element_type=jnp.float32)
    h = h + x * w1c_ref[...] + b1_ref[...]
    h = jnp.where(h >= 0, h, 0.2 * h)
    h2 = h + jnp.dot(h, wr_ref[...], preferred_element_type=jnp.float32) + br_ref[...]
    h2 = jnp.where(h2 >= 0, h2, 0.2 * h2)
    r = jnp.dot(h2, w2_ref[...], preferred_element_type=jnp.float32) + b2_ref[...]

    offs = jnp.tanh(r[:, :_K]) * (6.0 / _L)
    rw = r[:, _K:]
    m = jnp.max(rw, axis=1, keepdims=True)
    e = jnp.exp(rw - m)
    dw = e / jnp.sum(e, axis=1, keepdims=True)

    xk = x + offs
    ixk = _anchor_ix(xk)
    x0kf = jnp.floor(ixk)
    wk = ixk - x0kf
    x0k = x0kf.astype(jnp.int32)
    x1k = jnp.minimum(x0k + 1, _L - 1)

    x0a = x0f.astype(jnp.int32)
    wb = jnp.clip(x0a - 3, 0, _L - 8)
    p0 = jnp.clip(x0k - wb, 0, 7)
    p1 = jnp.clip(x1k - wb, 0, 7)

    jrow = lax.broadcasted_iota(jnp.int32, (1, 8), 1)
    ww = jnp.zeros(ww_ref.shape, jnp.float32)
    for k in range(_K):
        cw0 = dw[:, k:k + 1] * (1.0 - wk[:, k:k + 1])
        cw1 = dw[:, k:k + 1] * wk[:, k:k + 1]
        ww = ww + jnp.where(p0[:, k:k + 1] == jrow, cw0, 0.0)
        ww = ww + jnp.where(p1[:, k:k + 1] == jrow, cw1, 0.0)

    b = pl.program_id(0) // (_N // _PB)
    gb_ref[...] = wb + b * _L
    ww_ref[...] = ww


def _stage_b(f0, f1, xcol, w1aT, w1c, b1r, wrT, brr, w2T, b2r):
    grid = (_BN // _PB,)
    full = lambda shape: pl.BlockSpec(shape, lambda i: (0, 0))
    return pl.pallas_call(
        _router_body,
        grid=grid,
        in_specs=[
            pl.BlockSpec((_PB, _C), lambda i: (i, 0)),
            pl.BlockSpec((_PB, _C), lambda i: (i, 0)),
            pl.BlockSpec((_PB, 1), lambda i: (i, 0)),
            full((_C, _H)),
            full((1, _H)),
            full((1, _H)),
            full((_H, _H)),
            full((1, _H)),
            full((_H, 2 * _K)),
            full((1, 2 * _K)),
        ],
        out_specs=[
            pl.BlockSpec((_PB, 1), lambda i: (i, 0)),
            pl.BlockSpec((_PB, 8), lambda i: (i, 0)),
        ],
        out_shape=[
            jax.ShapeDtypeStruct((_BN, 1), jnp.int32),
            jax.ShapeDtypeStruct((_BN, 8), jnp.float32),
        ],
    )(f0, f1, xcol, w1aT, w1c, b1r, wrT, brr, w2T, b2r)


@functools.partial(
    pl.kernel,
    out_type=jax.ShapeDtypeStruct((_BN, _C), jnp.float32),
    mesh=_mesh,
    scratch_types=[
        pltpu.VMEM((_PPW,), jnp.int32),
        pltpu.VMEM((_PPW * 8,), jnp.float32),
        pltpu.VMEM((128,), jnp.int32),
        pltpu.VMEM((128, _C), jnp.float32),
        pltpu.VMEM((16, _C), jnp.float32),
        pltpu.SemaphoreType.DMA,
    ],
    compiler_params=_sc_params,
)
def _stage_c(gb_hbm, ww_hbm, featT_hbm, out_hbm,
             gb_v, ww_v, idx_v, buf_v, out_v, sem):
    base = _worker_base()
    pltpu.sync_copy(gb_hbm.at[pl.ds(base, _PPW)], gb_v)
    pltpu.sync_copy(ww_hbm.at[pl.ds(base * 8, _PPW * 8)], ww_v)
    lane = lax.iota(jnp.int32, 16)

    def round_body(g, carry):
        p0 = g * 16
        gv = gb_v[pl.ds(p0, 16)]
        for j in range(8):
            plsc.store_scatter(idx_v, [lane * 8 + j], gv + j)
        pltpu.async_copy(featT_hbm.at[idx_v], buf_v, sem).wait()
        wvecs = [
            plsc.load_gather(ww_v, [(p0 + lane) * 8 + j])
            for j in range(8)
        ]
        rowbase = lane * 8

        def cbody(c, c2):
            cf = jnp.broadcast_to(c, (16,))
            acc = jnp.zeros((16,), jnp.float32)
            for j in range(8):
                acc = acc + plsc.load_gather(buf_v, [rowbase + j, cf]) * wvecs[j]
            plsc.store_scatter(out_v, [lane, cf], acc)
            return c2

        lax.fori_loop(0, _C, cbody, 0)
        pltpu.sync_copy(out_v, out_hbm.at[pl.ds(base + p0, 16)])
        return carry

    lax.fori_loop(0, _PPW // 16, round_body, 0)


def kernel(feat_1d, coords_1d, W1, b1, Wr, br, W2, b2):
    assert feat_1d.shape == (_B, _C, _L) and coords_1d.shape == (_B, _N, 1)
    featT = jnp.transpose(feat_1d, (0, 2, 1)).reshape(_BL, _C)
    coords = coords_1d.reshape(_BN)
    f0, f1 = _stage_a(coords, featT)
    gb, ww = _stage_b(
        f0, f1, coords.reshape(_BN, 1),
        W1[:, :_C].T, W1[:, _C].reshape(1, _H), b1.reshape(1, _H),
        Wr.T, br.reshape(1, _H), W2.T, b2.reshape(1, 2 * _K),
    )
    out = _stage_c(gb.reshape(_BN), ww.reshape(_BN * 8), featT)
    return out.reshape(_B, _N, _C)


# speedup vs baseline: 18.3305x; 3.1109x over previous
import functools

import jax
import jax.numpy as jnp
from jax import lax
from jax.experimental import pallas as pl
from jax.experimental.pallas import tpu as pltpu
from jax.experimental.pallas import tpu_sc as plsc

_B, _C, _L, _N = 8, 128, 8192, 8192
_K = 5
_H = 64
_BN = _B * _N
_BL = _B * _L
_NW = 32
_PPW = _BN // _NW
_CHA = 128
_PB = 2048

_mesh = plsc.VectorSubcoreMesh(core_axis_name="c", subcore_axis_name="s")
_sc_params = pltpu.CompilerParams(needs_layout_passes=False)


def _worker_base():
    wid = lax.axis_index("s") * 2 + lax.axis_index("c")
    return wid * _PPW


def _anchor_ix(xv):
    return jnp.clip((xv + 1.0) * 0.5 * (_L - 1), 0.0, float(_L - 1))


@functools.partial(
    pl.kernel,
    out_type=(
        jax.ShapeDtypeStruct((_BN, _C), jnp.float32),
        jax.ShapeDtypeStruct((_BN, _C), jnp.float32),
    ),
    mesh=_mesh,
    scratch_types=[
        pltpu.VMEM((_PPW,), jnp.float32),
        pltpu.VMEM((_CHA,), jnp.int32),
        pltpu.VMEM((_CHA,), jnp.int32),
        pltpu.VMEM((_CHA, _C), jnp.float32),
        pltpu.VMEM((_CHA, _C), jnp.float32),
        pltpu.SemaphoreType.DMA,
        pltpu.SemaphoreType.DMA,
    ],
    compiler_params=_sc_params,
)
def _stage_a(coords_hbm, featT_hbm, f0_hbm, f1_hbm,
             coords_v, idx0_v, idx1_v, buf0_v, buf1_v, sem0, sem1):
    base = _worker_base()
    boff = (base // _N) * _L
    pltpu.sync_copy(coords_hbm.at[pl.ds(base, _PPW)], coords_v)

    def round_body(g, carry):
        rbase = g * _CHA
        for i in range(_CHA // 16):
            xv = coords_v[pl.ds(rbase + i * 16, 16)]
            ix = _anchor_ix(xv)
            x0 = ix.astype(jnp.int32)
            x1 = jnp.minimum(x0 + 1, _L - 1)
            idx0_v[pl.ds(i * 16, 16)] = x0 + boff
            idx1_v[pl.ds(i * 16, 16)] = x1 + boff
        cp0 = pltpu.async_copy(featT_hbm.at[idx0_v], buf0_v, sem0)
        cp1 = pltpu.async_copy(featT_hbm.at[idx1_v], buf1_v, sem1)
        cp0.wait()
        cp1.wait()
        pltpu.sync_copy(buf0_v, f0_hbm.at[pl.ds(base + rbase, _CHA)])
        pltpu.sync_copy(buf1_v, f1_hbm.at[pl.ds(base + rbase, _CHA)])
        return carry

    lax.fori_loop(0, _PPW // _CHA, round_body, 0)


def _router_body(f0_ref, f1_ref, xc_ref, w1a_ref, w1c_ref, b1_ref,
                 wr_ref, br_ref, w2_ref, b2_ref, gb_ref, ww_ref):
    x = xc_ref[...]
    ixa = _anchor_ix(x)
    x0f = jnp.floor(ixa)
    wa = ixa - x0f
    fa = f0_ref[...] * (1.0 - wa) + f1_ref[...] * wa

    h = jnp.dot(fa, w1a_ref[...], preferred_element_type=jnp.float32)
    h = h + x * w1c_ref[...] + b1_ref[...]
    h = jnp.where(h >= 0, h, 0.2 * h)
    h2 = h + jnp.dot(h, wr_ref[...], preferred_element_type=jnp.float32) + br_ref[...]
    h2 = jnp.where(h2 >= 0, h2, 0.2 * h2)
    r = jnp.dot(h2, w2_ref[...], preferred_element_type=jnp.float32) + b2_ref[...]

    offs = jnp.tanh(r[:, :_K]) * (6.0 / _L)
    rw = r[:, _K:]
    m = jnp.max(rw, axis=1, keepdims=True)
    e = jnp.exp(rw - m)
    dw = e / jnp.sum(e, axis=1, keepdims=True)

    xk = x + offs
    ixk = _anchor_ix(xk)
    x0kf = jnp.floor(ixk)
    wk = ixk - x0kf
    x0k = x0kf.astype(jnp.int32)
    x1k = jnp.minimum(x0k + 1, _L - 1)

    x0a = x0f.astype(jnp.int32)
    wb = jnp.clip(x0a - 3, 0, _L - 8)
    p0 = jnp.clip(x0k - wb, 0, 7)
    p1 = jnp.clip(x1k - wb, 0, 7)

    jrow = lax.broadcasted_iota(jnp.int32, (1, _C), 1) // 16
    ww = jnp.zeros(ww_ref.shape, jnp.float32)
    for k in range(_K):
        cw0 = dw[:, k:k + 1] * (1.0 - wk[:, k:k + 1])
        cw1 = dw[:, k:k + 1] * wk[:, k:k + 1]
        ww = ww + jnp.where(p0[:, k:k + 1] == jrow, cw0, 0.0)
        ww = ww + jnp.where(p1[:, k:k + 1] == jrow, cw1, 0.0)

    b = pl.program_id(0) // (_N // _PB)
    gb_ref[...] = wb + b * _L
    ww_ref[...] = ww


def _stage_b(f0, f1, xcol, w1aT, w1c, b1r, wrT, brr, w2T, b2r):
    grid = (_BN // _PB,)
    full = lambda shape: pl.BlockSpec(shape, lambda i: (0, 0))
    return pl.pallas_call(
        _router_body,
        grid=grid,
        in_specs=[
            pl.BlockSpec((_PB, _C), lambda i: (i, 0)),
            pl.BlockSpec((_PB, _C), lambda i: (i, 0)),
            pl.BlockSpec((_PB, 1), lambda i: (i, 0)),
            full((_C, _H)),
            full((1, _H)),
            full((1, _H)),
            full((_H, _H)),
            full((1, _H)),
            full((_H, 2 * _K)),
            full((1, 2 * _K)),
        ],
        out_specs=[
            pl.BlockSpec((_PB, 1), lambda i: (i, 0)),
            pl.BlockSpec((_PB, _C), lambda i: (i, 0)),
        ],
        out_shape=[
            jax.ShapeDtypeStruct((_BN, 1), jnp.int32),
            jax.ShapeDtypeStruct((_BN, _C), jnp.float32),
        ],
    )(f0, f1, xcol, w1aT, w1c, b1r, wrT, brr, w2T, b2r)


_CHC = 32
_RC = _PPW // _CHC


@functools.partial(
    pl.kernel,
    out_type=jax.ShapeDtypeStruct((_BN, _C), jnp.float32),
    mesh=_mesh,
    scratch_types=[
        pltpu.VMEM((_PPW,), jnp.int32),
        pltpu.VMEM((4 * 128,), jnp.int32),
        pltpu.VMEM((4, 128, _C), jnp.float32),
        pltpu.VMEM((2, _CHC, _C), jnp.float32),
        pltpu.VMEM((2, _CHC, _C), jnp.float32),
        pltpu.SemaphoreType.DMA,
        pltpu.SemaphoreType.DMA,
    ],
    compiler_params=_sc_params,
)
def _stage_c(gb_hbm, wwb_hbm, featT_hbm, out_hbm,
             gb_v, idx_v, buf_v, ww_v, out_v, sem0, sem1):
    base = _worker_base()
    pltpu.sync_copy(gb_hbm.at[pl.ds(base, _PPW)], gb_v)
    lane = lax.iota(jnp.int32, 16)
    lane8 = lane * 8
    sems = (sem0, sem1)

    def fire(g, s):
        sem = sems[s]
        for h in range(2):
            p0 = g * _CHC + h * 16
            gv = gb_v[pl.ds(p0, 16)]
            k = s * 2 + h
            for j in range(8):
                plsc.store_scatter(idx_v, [lane8 + (k * 128 + j)], gv + j)
            pltpu.async_copy(
                featT_hbm.at[idx_v.at[pl.ds(k * 128, 128)]], buf_v.at[k], sem)
        pltpu.async_copy(
            wwb_hbm.at[pl.ds(base + g * _CHC, _CHC)], ww_v.at[s], sem)

    def wait_slot(s):
        sem = sems[s]
        for h in range(2):
            k = s * 2 + h
            pltpu.make_async_copy(
                featT_hbm.at[idx_v.at[pl.ds(k * 128, 128)]], buf_v.at[k], sem
            ).wait()
        pltpu.make_async_copy(
            wwb_hbm.at[pl.ds(base, _CHC)], ww_v.at[0], sem).wait()

    def compute(g, s):
        for h in range(2):
            k = s * 2 + h

            def pbody(p2, c2):
                p = h * 16 + p2
                row = p2 * 8
                wvs = [ww_v[s, p, pl.ds(j * 16, 16)] for j in range(8)]
                for cv in range(8):
                    sl = pl.ds(cv * 16, 16)
                    t0 = wvs[0] * buf_v[k, row, sl] + wvs[1] * buf_v[k, row + 1, sl]
                    t1 = wvs[2] * buf_v[k, row + 2, sl] + wvs[3] * buf_v[k, row + 3, sl]
                    t2 = wvs[4] * buf_v[k, row + 4, sl] + wvs[5] * buf_v[k, row + 5, sl]
                    t3 = wvs[6] * buf_v[k, row + 6, sl] + wvs[7] * buf_v[k, row + 7, sl]
                    out_v[s, p, sl] = (t0 + t1) + (t2 + t3)
                return c2

            lax.fori_loop(0, 16, pbody, 0)
        pltpu.sync_copy(out_v.at[s], out_hbm.at[pl.ds(base + g * _CHC, _CHC)])

    fire(0, 0)
    fire(1, 1)

    def outer(t, carry):
        g0 = 2 * t
        wait_slot(0)
        compute(g0, 0)

        @pl.when(g0 + 2 < _RC)
        def _():
            fire(g0 + 2, 0)

        wait_slot(1)
        compute(g0 + 1, 1)

        @pl.when(g0 + 3 < _RC)
        def _():
            fire(g0 + 3, 1)

        return carry

    lax.fori_loop(0, _RC // 2, outer, 0)


def kernel(feat_1d, coords_1d, W1, b1, Wr, br, W2, b2):
    assert feat_1d.shape == (_B, _C, _L) and coords_1d.shape == (_B, _N, 1)
    featT = jnp.transpose(feat_1d, (0, 2, 1)).reshape(_BL, _C)
    coords = coords_1d.reshape(_BN)
    f0, f1 = _stage_a(coords, featT)
    gb, ww = _stage_b(
        f0, f1, coords.reshape(_BN, 1),
        W1[:, :_C].T, W1[:, _C].reshape(1, _H), b1.reshape(1, _H),
        Wr.T, br.reshape(1, _H), W2.T, b2.reshape(1, 2 * _K),
    )
    out = _stage_c(gb.reshape(_BN), ww, featT)
    return out.reshape(_B, _N, _C)


# speedup vs baseline: 30.1636x; 1.6455x over previous
import functools

import jax
import jax.numpy as jnp
from jax import lax
from jax.experimental import pallas as pl
from jax.experimental.pallas import tpu as pltpu
from jax.experimental.pallas import tpu_sc as plsc

_B, _C, _L, _N = 8, 128, 8192, 8192
_K = 5
_H = 64
_BN = _B * _N
_BL = _B * _L
_NW = 32
_PPW = _BN // _NW
_CHA = 128
_PB = 2048

_mesh = plsc.VectorSubcoreMesh(core_axis_name="c", subcore_axis_name="s")
_sc_params = pltpu.CompilerParams(needs_layout_passes=False)


def _worker_base():
    wid = lax.axis_index("s") * 2 + lax.axis_index("c")
    return wid * _PPW


def _anchor_ix(xv):
    return jnp.clip((xv + 1.0) * 0.5 * (_L - 1), 0.0, float(_L - 1))


@functools.partial(
    pl.kernel,
    out_type=(
        jax.ShapeDtypeStruct((_BN, _C), jnp.float32),
        jax.ShapeDtypeStruct((_BN, _C), jnp.float32),
    ),
    mesh=_mesh,
    scratch_types=[
        pltpu.VMEM((_PPW,), jnp.float32),
        pltpu.VMEM((_CHA,), jnp.int32),
        pltpu.VMEM((_CHA,), jnp.int32),
        pltpu.VMEM((_CHA, _C), jnp.float32),
        pltpu.VMEM((_CHA, _C), jnp.float32),
        pltpu.SemaphoreType.DMA,
        pltpu.SemaphoreType.DMA,
    ],
    compiler_params=_sc_params,
)
def _stage_a(coords_hbm, featT_hbm, f0_hbm, f1_hbm,
             coords_v, idx0_v, idx1_v, buf0_v, buf1_v, sem0, sem1):
    base = _worker_base()
    boff = (base // _N) * _L
    pltpu.sync_copy(coords_hbm.at[pl.ds(base, _PPW)], coords_v)

    def round_body(g, carry):
        rbase = g * _CHA
        for i in range(_CHA // 16):
            xv = coords_v[pl.ds(rbase + i * 16, 16)]
            ix = _anchor_ix(xv)
            x0 = ix.astype(jnp.int32)
            x1 = jnp.minimum(x0 + 1, _L - 1)
            idx0_v[pl.ds(i * 16, 16)] = x0 + boff
            idx1_v[pl.ds(i * 16, 16)] = x1 + boff
        cp0 = pltpu.async_copy(featT_hbm.at[idx0_v], buf0_v, sem0)
        cp1 = pltpu.async_copy(featT_hbm.at[idx1_v], buf1_v, sem1)
        cp0.wait()
        cp1.wait()
        pltpu.sync_copy(buf0_v, f0_hbm.at[pl.ds(base + rbase, _CHA)])
        pltpu.sync_copy(buf1_v, f1_hbm.at[pl.ds(base + rbase, _CHA)])
        return carry

    lax.fori_loop(0, _PPW // _CHA, round_body, 0)


def _dot(a, b):
    return lax.dot_general(a, b, (((1,), (0,)), ((), ())),
                           preferred_element_type=jnp.float32)


def _router_body(f0_ref, f1_ref, xr_ref, w1a_ref, w1cb_ref,
                 wre_ref, w2e_ref, gb_ref, ww_ref):
    xT = xr_ref[0]
    ixa = _anchor_ix(xT)
    x0f = jnp.floor(ixa)
    wa = ixa - x0f

    h0 = lax.dot_general(w1a_ref[...], f0_ref[...], (((1,), (1,)), ((), ())),
                         preferred_element_type=jnp.float32)
    h1 = lax.dot_general(w1a_ref[...], f1_ref[...], (((1,), (1,)), ((), ())),
                         preferred_element_type=jnp.float32)
    ones = jnp.ones_like(xT)
    x2 = jnp.concatenate([xT, ones], axis=0)
    h = h0 + wa * (h1 - h0) + _dot(w1cb_ref[...], x2)
    h = jnp.where(h >= 0, h, 0.2 * h)
    he = jnp.concatenate([h, ones], axis=0)
    h2 = h + _dot(wre_ref[...], he)
    h2 = jnp.where(h2 >= 0, h2, 0.2 * h2)
    h2e = jnp.concatenate([h2, ones], axis=0)
    rT = _dot(w2e_ref[...], h2e)

    offs = jnp.tanh(rT[:_K, :]) * (6.0 / _L)
    rw = rT[_K:, :]
    m = jnp.max(rw, axis=0, keepdims=True)
    e = jnp.exp(rw - m)
    dw = e / jnp.sum(e, axis=0, keepdims=True)

    xk = xT + offs
    ixk = _anchor_ix(xk)
    x0kf = jnp.floor(ixk)
    wk = ixk - x0kf
    x0k = x0kf.astype(jnp.int32)
    x1k = jnp.minimum(x0k + 1, _L - 1)

    x0a = x0f.astype(jnp.int32)
    wb = jnp.clip(x0a - 3, 0, _L - 8)
    p0 = jnp.clip(x0k - wb, 0, 7)
    p1 = jnp.clip(x1k - wb, 0, 7)
    cw0 = dw * (1.0 - wk)
    cw1 = dw * wk

    rows = []
    for j in range(8):
        rows.append(
            jnp.sum(jnp.where(p0 == j, cw0, 0.0), axis=0, keepdims=True)
            + jnp.sum(jnp.where(p1 == j, cw1, 0.0), axis=0, keepdims=True))
    wwT = jnp.concatenate(rows, axis=0)
    expand = (lax.broadcasted_iota(jnp.int32, (8, _C), 1) // 16
              == lax.broadcasted_iota(jnp.int32, (8, _C), 0)).astype(jnp.float32)
    wwb = lax.dot_general(wwT, expand, (((0,), (0,)), ((), ())),
                          preferred_element_type=jnp.float32)

    b = pl.program_id(0) // (_N // _PB)
    gb_ref[0] = wb + b * _L
    ww_ref[...] = wwb


def _stage_b(f0, f1, xrow, w1a, w1cb, wre, w2e):
    grid = (_BN // _PB,)
    full = lambda shape: pl.BlockSpec(shape, lambda i: (0, 0))
    return pl.pallas_call(
        _router_body,
        grid=grid,
        in_specs=[
            pl.BlockSpec((_PB, _C), lambda i: (i, 0)),
            pl.BlockSpec((_PB, _C), lambda i: (i, 0)),
            pl.BlockSpec((1, 1, _PB), lambda i: (i, 0, 0)),
            full((_H, _C)),
            full((_H, 2)),
            full((_H, _H + 1)),
            full((2 * _K, _H + 1)),
        ],
        out_specs=[
            pl.BlockSpec((1, 1, _PB), lambda i: (i, 0, 0)),
            pl.BlockSpec((_PB, _C), lambda i: (i, 0)),
        ],
        out_shape=[
            jax.ShapeDtypeStruct((_BN // _PB, 1, _PB), jnp.int32),
            jax.ShapeDtypeStruct((_BN, _C), jnp.float32),
        ],
    )(f0, f1, xrow, w1a, w1cb, wre, w2e)


_CHC = 32
_RC = _PPW // _CHC


@functools.partial(
    pl.kernel,
    out_type=jax.ShapeDtypeStruct((_BN, _C), jnp.float32),
    mesh=_mesh,
    scratch_types=[
        pltpu.VMEM((_PPW,), jnp.int32),
        pltpu.VMEM((4 * 128,), jnp.int32),
        pltpu.VMEM((4, 128, _C), jnp.float32),
        pltpu.VMEM((2, _CHC, _C), jnp.float32),
        pltpu.VMEM((2, _CHC, _C), jnp.float32),
        pltpu.SemaphoreType.DMA,
        pltpu.SemaphoreType.DMA,
    ],
    compiler_params=_sc_params,
)
def _stage_c(gb_hbm, wwb_hbm, featT_hbm, out_hbm,
             gb_v, idx_v, buf_v, ww_v, out_v, sem0, sem1):
    base = _worker_base()
    pltpu.sync_copy(gb_hbm.at[pl.ds(base, _PPW)], gb_v)
    lane = lax.iota(jnp.int32, 16)
    lane8 = lane * 8
    sems = (sem0, sem1)

    def fire(g, s):
        sem = sems[s]
        for h in range(2):
            p0 = g * _CHC + h * 16
            gv = gb_v[pl.ds(p0, 16)]
            k = s * 2 + h
            for j in range(8):
                plsc.store_scatter(idx_v, [lane8 + (k * 128 + j)], gv + j)
            pltpu.async_copy(
                featT_hbm.at[idx_v.at[pl.ds(k * 128, 128)]], buf_v.at[k], sem)
        pltpu.async_copy(
            wwb_hbm.at[pl.ds(base + g * _CHC, _CHC)], ww_v.at[s], sem)

    def wait_slot(s):
        sem = sems[s]
        for h in range(2):
            k = s * 2 + h
            pltpu.make_async_copy(
                featT_hbm.at[idx_v.at[pl.ds(k * 128, 128)]], buf_v.at[k], sem
            ).wait()
        pltpu.make_async_copy(
            wwb_hbm.at[pl.ds(base, _CHC)], ww_v.at[0], sem).wait()

    def compute(g, s):
        for h in range(2):
            k = s * 2 + h

            def pbody(p2, c2):
                p = h * 16 + p2
                row = p2 * 8
                wvs = [ww_v[s, p, pl.ds(j * 16, 16)] for j in range(8)]
                for cv in range(8):
                    sl = pl.ds(cv * 16, 16)
                    t0 = wvs[0] * buf_v[k, row, sl] + wvs[1] * buf_v[k, row + 1, sl]
                    t1 = wvs[2] * buf_v[k, row + 2, sl] + wvs[3] * buf_v[k, row + 3, sl]
                    t2 = wvs[4] * buf_v[k, row + 4, sl] + wvs[5] * buf_v[k, row + 5, sl]
                    t3 = wvs[6] * buf_v[k, row + 6, sl] + wvs[7] * buf_v[k, row + 7, sl]
                    out_v[s, p, sl] = (t0 + t1) + (t2 + t3)
                return c2

            lax.fori_loop(0, 16, pbody, 0)
        pltpu.sync_copy(out_v.at[s], out_hbm.at[pl.ds(base + g * _CHC, _CHC)])

    fire(0, 0)
    fire(1, 1)

    def outer(t, carry):
        g0 = 2 * t
        wait_slot(0)
        compute(g0, 0)

        @pl.when(g0 + 2 < _RC)
        def _():
            fire(g0 + 2, 0)

        wait_slot(1)
        compute(g0 + 1, 1)

        @pl.when(g0 + 3 < _RC)
        def _():
            fire(g0 + 3, 1)

        return carry

    lax.fori_loop(0, _RC // 2, outer, 0)


def kernel(feat_1d, coords_1d, W1, b1, Wr, br, W2, b2):
    assert feat_1d.shape == (_B, _C, _L) and coords_1d.shape == (_B, _N, 1)
    featT = jnp.transpose(feat_1d, (0, 2, 1)).reshape(_BL, _C)
    coords = coords_1d.reshape(_BN)
    f0, f1 = _stage_a(coords, featT)
    gb, ww = _stage_b(
        f0, f1, coords.reshape(_BN // _PB, 1, _PB),
        W1[:, :_C],
        jnp.stack([W1[:, _C], b1], axis=1),
        jnp.concatenate([Wr, br[:, None]], axis=1),
        jnp.concatenate([W2, b2[:, None]], axis=1),
    )
    out = _stage_c(gb.reshape(_BN), ww, featT)
    return out.reshape(_B, _N, _C)
